# baseline (speedup 1.0000x reference)
import functools

import jax
import jax.numpy as jnp
from jax import lax
from jax.experimental import pallas as pl
from jax.experimental.pallas import tpu as pltpu
from jax.experimental.pallas import tpu_sc as plsc

B = 16384
GMF_D = 8
MLP_D = 16
CHUNK = 128


def _gather_sc(user2d, item2d, gmf_user_emb, gmf_item_emb, mlp_user_emb, mlp_item_emb):
    info = plsc.get_sparse_core_info()
    NC, NS = info.num_cores, info.num_subcores
    NW = NC * NS
    n_rows = B // CHUNK
    rows_per_w = n_rows // NW

    mesh = plsc.VectorSubcoreMesh(core_axis_name="c", subcore_axis_name="s")

    @functools.partial(
        pl.kernel,
        mesh=mesh,
        compiler_params=pltpu.CompilerParams(use_tc_tiling_on_sc=False),
        out_type=[
            jax.ShapeDtypeStruct((n_rows, CHUNK, GMF_D), jnp.float32),
            jax.ShapeDtypeStruct((n_rows, CHUNK, GMF_D), jnp.float32),
            jax.ShapeDtypeStruct((n_rows, CHUNK, MLP_D), jnp.float32),
            jax.ShapeDtypeStruct((n_rows, CHUNK, MLP_D), jnp.float32),
        ],
        scratch_types=[
            pltpu.VMEM((rows_per_w, CHUNK), jnp.int32),
            pltpu.VMEM((rows_per_w, CHUNK), jnp.int32),
            pltpu.VMEM((rows_per_w, CHUNK, GMF_D), jnp.float32),
            pltpu.VMEM((rows_per_w, CHUNK, GMF_D), jnp.float32),
            pltpu.VMEM((rows_per_w, CHUNK, MLP_D), jnp.float32),
            pltpu.VMEM((rows_per_w, CHUNK, MLP_D), jnp.float32),
            pltpu.SemaphoreType.DMA,
        ],
    )
    def gather_kernel(user_hbm, item_hbm, gu_tab, gi_tab, mu_tab, mi_tab,
                      gu_out, gi_out, mu_out, mi_out,
                      uidx, iidx, gu_v, gi_v, mu_v, mi_v, sem):
        wid = lax.axis_index("s") * NC + lax.axis_index("c")
        row0 = wid * rows_per_w
        pltpu.sync_copy(user_hbm.at[pl.ds(row0, rows_per_w)], uidx)
        pltpu.sync_copy(item_hbm.at[pl.ds(row0, rows_per_w)], iidx)
        copies = []
        for r in range(rows_per_w):
            copies.append(pltpu.async_copy(gu_tab.at[uidx.at[r]], gu_v.at[r], sem))
            copies.append(pltpu.async_copy(gi_tab.at[iidx.at[r]], gi_v.at[r], sem))
            copies.append(pltpu.async_copy(mu_tab.at[uidx.at[r]], mu_v.at[r], sem))
            copies.append(pltpu.async_copy(mi_tab.at[iidx.at[r]], mi_v.at[r], sem))
        for c in copies:
            c.wait()
        pltpu.sync_copy(gu_v, gu_out.at[pl.ds(row0, rows_per_w)])
        pltpu.sync_copy(gi_v, gi_out.at[pl.ds(row0, rows_per_w)])
        pltpu.sync_copy(mu_v, mu_out.at[pl.ds(row0, rows_per_w)])
        pltpu.sync_copy(mi_v, mi_out.at[pl.ds(row0, rows_per_w)])

    return gather_kernel(user2d, item2d, gmf_user_emb, gmf_item_emb,
                         mlp_user_emb, mlp_item_emb)


def _head_tc_body(gu, gi, mu, mi, w1u, w1i, b1, w2, b2, wlg, wlh, bl, out):
    gmf = gu[...] * gi[...]
    h = mu[...] @ w1u[...] + mi[...] @ w1i[...] + b1[...]
    h = jnp.maximum(h, 0.0)
    h = h @ w2[...] + b2[...]
    h = jnp.maximum(h, 0.0)
    logits = gmf @ wlg[...] + h @ wlh[...] + bl[...]
    out[...] = jax.nn.sigmoid(logits)


def kernel(user, item, gmf_user_emb, gmf_item_emb, mlp_user_emb, mlp_item_emb,
           W1, b1, W2, b2, Wl, bl):
    user2d = user.astype(jnp.int32).reshape(B // CHUNK, CHUNK)
    item2d = item.astype(jnp.int32).reshape(B // CHUNK, CHUNK)

    gu3, gi3, mu3, mi3 = _gather_sc(user2d, item2d, gmf_user_emb, gmf_item_emb,
                                    mlp_user_emb, mlp_item_emb)
    gu = gu3.reshape(B, GMF_D)
    gi = gi3.reshape(B, GMF_D)
    mu = mu3.reshape(B, MLP_D)
    mi = mi3.reshape(B, MLP_D)

    w1u = W1[:MLP_D]
    w1i = W1[MLP_D:]
    wlg = Wl[:GMF_D]
    wlh = Wl[GMF_D:]
    b1r = b1.reshape(1, -1)
    b2r = b2.reshape(1, -1)
    blr = bl.reshape(1, 1)

    out = pl.pallas_call(
        _head_tc_body,
        out_shape=jax.ShapeDtypeStruct((B, 1), jnp.float32),
    )(gu, gi, mu, mi, w1u, w1i, b1r, W2, b2r, wlg, wlh, blr)
    return out.reshape(-1)
